# Initial kernel scaffold; baseline (speedup 1.0000x reference)
import jax
import jax.numpy as jnp
from jax.experimental import pallas as pl


def kernel(target_features, source_features, select_mat, W, b):
    raise NotImplementedError("write your pallas kernel here")


# speedup vs baseline: 91.1726x; 91.1726x over previous
import jax
import jax.numpy as jnp
from jax.experimental import pallas as pl

_GC = 8


def _mp_kernel(t_ref, s_ref, sel_ref, w_ref, b_ref, o_ref):
    fea = t_ref.shape[1]
    gate_w = w_ref.shape[0]
    T = t_ref[:]
    S = s_ref[:]
    rT = jnp.maximum(T, 0.0)
    rS = jnp.maximum(S, 0.0)
    Wt = w_ref[:, :fea]
    Ws = w_ref[:, fea:]
    A = jax.lax.dot_general(rT, Wt, (((1,), (1,)), ((), ())),
                            preferred_element_type=jnp.float32) + b_ref[:]
    BT = jax.lax.dot_general(Ws, rS, (((1,), (1,)), ((), ())),
                             preferred_element_type=jnp.float32)
    nt = T.shape[0]
    ns = S.shape[0]
    acc = jnp.zeros((nt, ns), jnp.float32)
    for gi in range(0, gate_w, _GC):
        Ac = A[:, gi:gi + _GC]
        Bc = BT[gi:gi + _GC, :]
        acc = acc + jnp.sum(
            jax.nn.sigmoid(Ac[:, :, None] + Bc[None, :, :]), axis=1)
    sel = sel_ref[:] > 0
    counts = jnp.sum(jnp.where(sel, 1.0, 0.0), axis=1, keepdims=True)
    G = jnp.where(sel, acc, 0.0) * (1.0 / gate_w)
    out = jnp.dot(G, S, preferred_element_type=jnp.float32)
    o_ref[:] = jnp.where(counts > 0.0, out / jnp.maximum(counts, 1.0), 0.0)


def kernel(target_features, source_features, select_mat, W, b):
    return pl.pallas_call(
        _mp_kernel,
        out_shape=jax.ShapeDtypeStruct(
            (target_features.shape[0], source_features.shape[1]), jnp.float32),
    )(target_features, source_features, select_mat, W, b.reshape(1, -1))
